# baseline (speedup 1.0000x reference)
import functools

import jax
import jax.numpy as jnp
import numpy as np
from jax import lax
from jax.experimental import pallas as pl
from jax.experimental.pallas import tpu as pltpu
from jax.experimental.pallas import tpu_sc as plsc

_NC = 2
_NS = 16
_L = 16

_CHUNK = 128
_NCH = 20
_STRIPE = 624
_PSTRIPE = 1000

_COLPERM = np.concatenate(
    [np.concatenate([np.arange(0, 32, 2), np.arange(1, 32, 2)]) + 32 * c
     for c in range(4)])


def _sc_accumulate(xb, src_p, tgt_p, w_p, r_total, n):
  d = xb.shape[1] * 2
  npad = n + _L
  rel_per_core = r_total // _NC
  tail = n - _STRIPE * _NS

  mesh = plsc.VectorSubcoreMesh(core_axis_name="c", subcore_axis_name="s")

  @functools.partial(
      pl.kernel,
      out_type=(
          jax.ShapeDtypeStruct((r_total, npad, d), jnp.float32),
          jax.ShapeDtypeStruct((r_total * npad,), jnp.float32),
      ),
      mesh=mesh,
      compiler_params=pltpu.CompilerParams(use_tc_tiling_on_sc=False),
      scratch_types=[
          pltpu.VMEM((_NCH, _CHUNK), jnp.int32),
          pltpu.VMEM((_NCH, _CHUNK), jnp.int32),
          pltpu.VMEM((_NCH, _CHUNK), jnp.float32),
          pltpu.VMEM((_CHUNK, d // 2), jnp.int32),
          pltpu.VMEM((_CHUNK, d // 2), jnp.int32),
          pltpu.VMEM((_CHUNK, d), jnp.float32),
          pltpu.VMEM((_CHUNK,), jnp.float32),
          pltpu.VMEM((_PSTRIPE + 8,), jnp.float32),
          pltpu.VMEM((_PSTRIPE,), jnp.float32),
          pltpu.VMEM_SHARED((npad, d), jnp.float32),
          pltpu.VMEM_SHARED((npad,), jnp.float32),
          pltpu.SemaphoreType.DMA,
          pltpu.SemaphoreType.DMA,
          pltpu.SemaphoreType.DMA,
          pltpu.SemaphoreType.DMA,
      ],
  )
  def sc_kernel(xb_hbm, src_hbm, tgt_hbm, ew_hbm, a_hbm, p_hbm,
                src_v, tgt_v, w_v, rows0_v, rows1_v, pay_v, ones_v, z1_v,
                pv_v, a_sh, p_sh, gsem0, gsem1, ssem, psem):
    cid = lax.axis_index("c")
    sid = lax.axis_index("s")
    rows = (rows0_v, rows1_v)
    gsem = (gsem0, gsem1)

    one16 = jnp.full((_L,), 1.0, jnp.float32)
    zero16 = jnp.zeros((_L,), jnp.float32)
    for j in range(_CHUNK // _L):
      ones_v[pl.ds(j * _L, _L)] = one16

    def z1_body(i, carry):
      z1_v[pl.ds(i * _L, _L)] = zero16
      return carry
    lax.fori_loop(0, (_PSTRIPE + 8) // _L, z1_body, 0)

    def scale_chunk(buf, c):
      def jbody(j, carry):
        w16 = w_v[c, pl.ds(j * _L, _L)]
        for k in range(_L):
          w = w16[k]
          e = j * _L + k
          for blk in range(d // 32):
            v = buf[e, pl.ds(blk * _L, _L)]
            even = lax.bitcast_convert_type(v << 16, jnp.float32)
            odd = lax.bitcast_convert_type((v >> 16) << 16, jnp.float32)
            pay_v[e, pl.ds(blk * 32, _L)] = even * w
            pay_v[e, pl.ds(blk * 32 + _L, _L)] = odd * w
        return carry
      lax.fori_loop(0, _CHUNK // _L, jbody, 0)

    def rel_body(rr, carry):
      r = cid * rel_per_core + rr
      rt = r * _NS + sid

      def zfill_body(i, carry):
        for c in range(d // _L):
          pay_v[i, pl.ds(c * _L, _L)] = zero16
        return carry
      lax.fori_loop(0, _CHUNK, zfill_body, 0)

      for i in range(_STRIPE // _CHUNK):
        pltpu.sync_copy(pay_v,
                        a_sh.at[pl.ds(sid * _STRIPE + i * _CHUNK, _CHUNK)])
      rem = _STRIPE - (_STRIPE // _CHUNK) * _CHUNK
      pltpu.sync_copy(
          pay_v.at[pl.ds(0, rem)],
          a_sh.at[pl.ds(sid * _STRIPE + _STRIPE - rem, rem)])

      @pl.when(sid < 2)
      def _zero_tail():
        half = tail // 2
        pltpu.sync_copy(pay_v.at[pl.ds(0, half)],
                        a_sh.at[pl.ds(_STRIPE * _NS + sid * half, half)])

      @pl.when(sid < n // _PSTRIPE)
      def _zero_counts():
        pltpu.sync_copy(z1_v.at[pl.ds(0, _PSTRIPE)],
                        p_sh.at[pl.ds(sid * _PSTRIPE, _PSTRIPE)])

      plsc.subcore_barrier()

      pltpu.sync_copy(src_hbm.at[rt], src_v)
      pltpu.sync_copy(tgt_hbm.at[rt], tgt_v)
      pltpu.sync_copy(ew_hbm.at[rt], w_v)

      pltpu.async_copy(xb_hbm.at[src_v.at[0]], rows0_v, gsem0)

      def chunk_body(s, carry):
        for b in range(2):
          c = s * 2 + b
          nxt = 1 - b

          def start_next():
            pltpu.async_copy(xb_hbm.at[src_v.at[c + 1]], rows[nxt],
                             gsem[nxt])
          if b == 0:
            start_next()
          else:
            pl.when(s < _NCH // 2 - 1)(start_next)

          pltpu.make_async_copy(xb_hbm.at[src_v.at[c]], rows[b],
                                gsem[b]).wait()

          def wait_prev():
            pltpu.make_async_copy(
                pay_v, a_sh.at[tgt_v.at[c]], ssem).wait()
            pltpu.make_async_copy(
                ones_v, p_sh.at[tgt_v.at[c]], psem).wait()
          if b == 0:
            pl.when(s >= 1)(wait_prev)
          else:
            wait_prev()

          scale_chunk(rows[b], c)
          pltpu.async_copy(pay_v, a_sh.at[tgt_v.at[c]], ssem, add=True)
          pltpu.async_copy(ones_v, p_sh.at[tgt_v.at[c]], psem, add=True)
        return carry
      lax.fori_loop(0, _NCH // 2, chunk_body, 0)

      pltpu.make_async_copy(pay_v, a_sh.at[tgt_v.at[_NCH - 1]], ssem).wait()
      pltpu.make_async_copy(ones_v, p_sh.at[tgt_v.at[_NCH - 1]], psem).wait()

      plsc.subcore_barrier()

      pltpu.sync_copy(a_sh.at[pl.ds(sid * _STRIPE, _STRIPE)],
                      a_hbm.at[r, pl.ds(sid * _STRIPE, _STRIPE)])

      @pl.when(sid < 2)
      def _write_tail():
        half = tail // 2
        pltpu.sync_copy(a_sh.at[pl.ds(_STRIPE * _NS + sid * half, half)],
                        a_hbm.at[r, pl.ds(_STRIPE * _NS + sid * half, half)])

      @pl.when(sid < n // _PSTRIPE)
      def _write_counts():
        pbase = pl.multiple_of(r * npad + sid * _PSTRIPE, 8)
        pltpu.sync_copy(p_sh.at[pl.ds(sid * _PSTRIPE, _PSTRIPE)], pv_v)
        pltpu.sync_copy(pv_v, p_hbm.at[pl.ds(pbase, _PSTRIPE)])

      plsc.subcore_barrier()
      return carry

    lax.fori_loop(0, rel_per_core, rel_body, 0)

  return sc_kernel(xb, src_p, tgt_p, w_p)


def _tc_combine(a, counts_t, x, rw_perm, self_weight, bias_param):
  n, d = x.shape
  r_total = rw_perm.shape[0]
  blk = 1000

  def body(a_ref, p_ref, x_ref, rw_ref, sw_ref, b_ref, o_ref):
    acc = jnp.dot(x_ref[...], sw_ref[...], preferred_element_type=jnp.float32)
    for r in range(r_total):
      acc = acc + jnp.dot(a_ref[r], rw_ref[r],
                          preferred_element_type=jnp.float32)
    present = (p_ref[...] > 0).astype(jnp.float32)
    acc = acc + jnp.dot(present, b_ref[...],
                        preferred_element_type=jnp.float32)
    o_ref[...] = acc

  return pl.pallas_call(
      body,
      grid=(n // blk,),
      in_specs=[
          pl.BlockSpec((r_total, blk, d), lambda i: (0, i, 0)),
          pl.BlockSpec((blk, r_total), lambda i: (i, 0)),
          pl.BlockSpec((blk, d), lambda i: (i, 0)),
          pl.BlockSpec((r_total, d, d), lambda i: (0, 0, 0)),
          pl.BlockSpec((d, d), lambda i: (0, 0)),
          pl.BlockSpec((r_total, d), lambda i: (0, 0)),
      ],
      out_specs=pl.BlockSpec((blk, d), lambda i: (i, 0)),
      out_shape=jax.ShapeDtypeStruct((n, d), jnp.float32),
  )(a, counts_t, x, rw_perm, self_weight, bias_param)


def kernel(entity_embeddings, edge_index, edge_weights, relation_weights,
           self_weight, bias_param):
  r_total, _, e_total = edge_index.shape
  n = entity_embeddings.shape[0]
  npad = n + _L
  ept = e_total // _NS
  pad = _NCH * _CHUNK - ept

  din = entity_embeddings.shape[1]
  xb = lax.bitcast_convert_type(
      entity_embeddings.astype(jnp.bfloat16).reshape(n, din // 2, 2),
      jnp.int32)
  src3 = edge_index[:, 0, :].reshape(r_total, _NS, ept)
  tgt3 = edge_index[:, 1, :].reshape(r_total, _NS, ept)
  w3 = edge_weights.reshape(r_total, _NS, ept)
  src_p = jnp.pad(src3, ((0, 0), (0, 0), (0, pad))).reshape(
      r_total * _NS, _NCH, _CHUNK)
  tgt_p = jnp.pad(tgt3, ((0, 0), (0, 0), (0, pad)),
                  constant_values=n).reshape(r_total * _NS, _NCH, _CHUNK)
  w_p = jnp.pad(w3, ((0, 0), (0, 0), (0, pad))).reshape(
      r_total * _NS, _NCH, _CHUNK)
  rw_perm = relation_weights[:, _COLPERM, :]

  a, counts = _sc_accumulate(xb, src_p, tgt_p, w_p, r_total, n)
  counts_t = counts.reshape(r_total, npad)[:, :n].T
  return _tc_combine(a, counts_t, entity_embeddings, rw_perm,
                     self_weight, bias_param)


# speedup vs baseline: 1.1093x; 1.1093x over previous
import functools

import jax
import jax.numpy as jnp
import numpy as np
from jax import lax
from jax.experimental import pallas as pl
from jax.experimental.pallas import tpu as pltpu
from jax.experimental.pallas import tpu_sc as plsc

_NC = 2
_NS = 16
_L = 16

_CHUNK = 32
_GRP = 10
_NGRP = 8
_STRIPE = 624
_PZ = 504

_COLPERM = np.concatenate(
    [np.concatenate([np.arange(0, 32, 2), np.arange(1, 32, 2)]) + 32 * c
     for c in range(4)])


def _sc_accumulate(xb, src_p, tgt_p, w_p, r_total, n):
  d = xb.shape[1] * 2
  npad = n + 8
  rel_per_core = r_total // _NC
  tail = npad - _STRIPE * _NS

  mesh = plsc.VectorSubcoreMesh(core_axis_name="c", subcore_axis_name="s")

  @functools.partial(
      pl.kernel,
      out_type=(
          jax.ShapeDtypeStruct((r_total, npad, d), jnp.float32),
          jax.ShapeDtypeStruct((r_total * npad,), jnp.float32),
      ),
      mesh=mesh,
      compiler_params=pltpu.CompilerParams(use_tc_tiling_on_sc=False),
      scratch_types=[
          pltpu.VMEM((_GRP, _CHUNK), jnp.int32),
          pltpu.VMEM((_GRP, _CHUNK), jnp.int32),
          pltpu.VMEM((_GRP, _CHUNK), jnp.float32),
          pltpu.VMEM((_CHUNK, d // 2), jnp.int32),
          pltpu.VMEM((_CHUNK, d // 2), jnp.int32),
          pltpu.VMEM((_CHUNK, d), jnp.float32),
          pltpu.VMEM((_CHUNK,), jnp.float32),
          pltpu.VMEM((_PZ,), jnp.float32),
          pltpu.VMEM((_PZ,), jnp.float32),
          pltpu.VMEM_SHARED((n, d // 2), jnp.int32),
          pltpu.VMEM_SHARED((npad, d), jnp.float32),
          pltpu.VMEM_SHARED((npad,), jnp.float32),
          pltpu.SemaphoreType.DMA,
          pltpu.SemaphoreType.DMA,
          pltpu.SemaphoreType.DMA,
          pltpu.SemaphoreType.DMA,
      ],
  )
  def sc_kernel(xb_hbm, src_hbm, tgt_hbm, ew_hbm, a_hbm, p_hbm,
                src_g, tgt_g, w_g, rows0_v, rows1_v, pay_v, ones_v, z1_v,
                pv_v, x_sh, a_sh, p_sh, gsem0, gsem1, ssem, psem):
    cid = lax.axis_index("c")
    sid = lax.axis_index("s")
    rows = (rows0_v, rows1_v)
    gsem = (gsem0, gsem1)

    one16 = jnp.full((_L,), 1.0, jnp.float32)
    zero16 = jnp.zeros((_L,), jnp.float32)
    for j in range(_CHUNK // _L):
      ones_v[pl.ds(j * _L, _L)] = one16

    def z1_body(i, carry):
      z1_v[pl.ds(i * _L, _L)] = zero16
      return carry
    lax.fori_loop(0, _PZ // _L, z1_body, 0)
    z1_v[pl.ds(_PZ - _L, _L)] = zero16

    pltpu.sync_copy(xb_hbm.at[pl.ds(sid * _STRIPE, _STRIPE)],
                    x_sh.at[pl.ds(sid * _STRIPE, _STRIPE)])

    @pl.when(sid < 2)
    def _stage_tail():
      half = (n - _STRIPE * _NS) // 2
      pltpu.sync_copy(xb_hbm.at[pl.ds(_STRIPE * _NS + sid * half, half)],
                      x_sh.at[pl.ds(_STRIPE * _NS + sid * half, half)])

    def scale_chunk(buf, w16s):
      def jbody(j, carry):
        w16 = w16s[pl.ds(j * _L, _L)]
        for k in range(_L):
          w = w16[k]
          e = j * _L + k
          for blk in range(d // 32):
            v = buf[e, pl.ds(blk * _L, _L)]
            even = lax.bitcast_convert_type(v << 16, jnp.float32)
            odd = lax.bitcast_convert_type((v >> 16) << 16, jnp.float32)
            pay_v[e, pl.ds(blk * 32, _L)] = even * w
            pay_v[e, pl.ds(blk * 32 + _L, _L)] = odd * w
        return carry
      lax.fori_loop(0, _CHUNK // _L, jbody, 0)

    def rel_body(rr, carry):
      r = cid * rel_per_core + rr
      rt = r * _NS + sid

      def zfill_body(i, carry):
        for c in range(d // _L):
          pay_v[i, pl.ds(c * _L, _L)] = zero16
        return carry
      lax.fori_loop(0, _CHUNK, zfill_body, 0)

      for i in range(_STRIPE // _CHUNK):
        pltpu.sync_copy(pay_v,
                        a_sh.at[pl.ds(sid * _STRIPE + i * _CHUNK, _CHUNK)])
      rem = _STRIPE - (_STRIPE // _CHUNK) * _CHUNK
      pltpu.sync_copy(
          pay_v.at[pl.ds(0, rem)],
          a_sh.at[pl.ds(sid * _STRIPE + _STRIPE - rem, rem)])

      @pl.when(sid < tail // 8)
      def _zero_tail():
        pltpu.sync_copy(pay_v.at[pl.ds(0, 8)],
                        a_sh.at[pl.ds(_STRIPE * _NS + sid * 8, 8)])

      pltpu.sync_copy(z1_v, p_sh.at[pl.ds(sid * _PZ, _PZ)])

      @pl.when(sid < 3)
      def _zero_counts2():
        pltpu.sync_copy(z1_v, p_sh.at[pl.ds((_NS + sid) * _PZ, _PZ)])

      @pl.when(sid == 3)
      def _zero_counts3():
        last = npad - 19 * _PZ
        pltpu.sync_copy(z1_v.at[pl.ds(0, last)],
                        p_sh.at[pl.ds(19 * _PZ, last)])

      plsc.subcore_barrier()

      def grp_body(g, carry):
        pltpu.sync_copy(src_hbm.at[rt, g], src_g)
        pltpu.sync_copy(tgt_hbm.at[rt, g], tgt_g)
        pltpu.sync_copy(ew_hbm.at[rt, g], w_g)

        pltpu.async_copy(x_sh.at[src_g.at[0]], rows0_v, gsem0)

        def chunk_body(s, carry2):
          for b in range(2):
            c = s * 2 + b
            nxt = 1 - b

            def start_next():
              pltpu.async_copy(x_sh.at[src_g.at[c + 1]], rows[nxt],
                               gsem[nxt])
            if b == 0:
              start_next()
            else:
              pl.when(s < _GRP // 2 - 1)(start_next)

            pltpu.make_async_copy(x_sh.at[src_g.at[c]], rows[b],
                                  gsem[b]).wait()

            def wait_prev():
              pltpu.make_async_copy(
                  pay_v, a_sh.at[tgt_g.at[c]], ssem).wait()
              pltpu.make_async_copy(
                  ones_v, p_sh.at[tgt_g.at[c]], psem).wait()
            if b == 0:
              pl.when(s >= 1)(wait_prev)
            else:
              wait_prev()

            scale_chunk(rows[b], w_g.at[c])
            pltpu.async_copy(pay_v, a_sh.at[tgt_g.at[c]], ssem, add=True)
            pltpu.async_copy(ones_v, p_sh.at[tgt_g.at[c]], psem, add=True)
          return carry2
        lax.fori_loop(0, _GRP // 2, chunk_body, 0)

        pltpu.make_async_copy(pay_v, a_sh.at[tgt_g.at[_GRP - 1]],
                              ssem).wait()
        pltpu.make_async_copy(ones_v, p_sh.at[tgt_g.at[_GRP - 1]],
                              psem).wait()
        return carry
      lax.fori_loop(0, _NGRP, grp_body, 0)

      plsc.subcore_barrier()

      pltpu.sync_copy(a_sh.at[pl.ds(sid * _STRIPE, _STRIPE)],
                      a_hbm.at[r, pl.ds(sid * _STRIPE, _STRIPE)])

      @pl.when(sid < tail // 8)
      def _write_tail():
        pltpu.sync_copy(a_sh.at[pl.ds(_STRIPE * _NS + sid * 8, 8)],
                        a_hbm.at[r, pl.ds(_STRIPE * _NS + sid * 8, 8)])

      def wb_counts(stripe, size):
        pltpu.sync_copy(p_sh.at[pl.ds(stripe * _PZ, size)],
                        pv_v.at[pl.ds(0, size)])
        pbase = pl.multiple_of(r * npad + stripe * _PZ, 8)
        pltpu.sync_copy(pv_v.at[pl.ds(0, size)],
                        p_hbm.at[pl.ds(pbase, size)])

      wb_counts(sid, _PZ)

      @pl.when(sid < 3)
      def _wb_counts2():
        wb_counts(_NS + sid, _PZ)

      @pl.when(sid == 3)
      def _wb_counts3():
        wb_counts(19, npad - 19 * _PZ)

      plsc.subcore_barrier()
      return carry

    lax.fori_loop(0, rel_per_core, rel_body, 0)

  return sc_kernel(xb, src_p, tgt_p, w_p)


def _tc_combine(a, counts_t, x, rw_perm, self_weight, bias_param):
  n, d = x.shape
  r_total = rw_perm.shape[0]
  blk = 1000

  def body(a_ref, p_ref, x_ref, rw_ref, sw_ref, b_ref, o_ref):
    acc = jnp.dot(x_ref[...], sw_ref[...], preferred_element_type=jnp.float32)
    for r in range(r_total):
      acc = acc + jnp.dot(a_ref[r], rw_ref[r],
                          preferred_element_type=jnp.float32)
    present = (p_ref[...] > 0).astype(jnp.float32)
    acc = acc + jnp.dot(present, b_ref[...],
                        preferred_element_type=jnp.float32)
    o_ref[...] = acc

  return pl.pallas_call(
      body,
      grid=(n // blk,),
      in_specs=[
          pl.BlockSpec((r_total, blk, d), lambda i: (0, i, 0)),
          pl.BlockSpec((blk, r_total), lambda i: (i, 0)),
          pl.BlockSpec((blk, d), lambda i: (i, 0)),
          pl.BlockSpec((r_total, d, d), lambda i: (0, 0, 0)),
          pl.BlockSpec((d, d), lambda i: (0, 0)),
          pl.BlockSpec((r_total, d), lambda i: (0, 0)),
      ],
      out_specs=pl.BlockSpec((blk, d), lambda i: (i, 0)),
      out_shape=jax.ShapeDtypeStruct((n, d), jnp.float32),
  )(a, counts_t, x, rw_perm, self_weight, bias_param)


def kernel(entity_embeddings, edge_index, edge_weights, relation_weights,
           self_weight, bias_param):
  r_total, _, e_total = edge_index.shape
  n, din = entity_embeddings.shape
  npad = n + 8
  ept = e_total // _NS
  pad = _NGRP * _GRP * _CHUNK - ept

  xb = lax.bitcast_convert_type(
      entity_embeddings.astype(jnp.bfloat16).reshape(n, din // 2, 2),
      jnp.int32)
  src3 = edge_index[:, 0, :].reshape(r_total, _NS, ept)
  tgt3 = edge_index[:, 1, :].reshape(r_total, _NS, ept)
  w3 = edge_weights.reshape(r_total, _NS, ept)
  shp = (r_total * _NS, _NGRP, _GRP, _CHUNK)
  src_p = jnp.pad(src3, ((0, 0), (0, 0), (0, pad))).reshape(shp)
  tgt_p = jnp.pad(tgt3, ((0, 0), (0, 0), (0, pad)),
                  constant_values=n).reshape(shp)
  w_p = jnp.pad(w3, ((0, 0), (0, 0), (0, pad))).reshape(shp)
  rw_perm = relation_weights[:, _COLPERM, :]

  a, counts = _sc_accumulate(xb, src_p, tgt_p, w_p, r_total, n)
  counts_t = counts.reshape(r_total, npad)[:, :n].T
  return _tc_combine(a, counts_t, entity_embeddings, rw_perm,
                     self_weight, bias_param)


# speedup vs baseline: 1.4280x; 1.2873x over previous
import functools

import jax
import jax.numpy as jnp
import numpy as np
from jax import lax
from jax.experimental import pallas as pl
from jax.experimental.pallas import tpu as pltpu
from jax.experimental.pallas import tpu_sc as plsc

_NC = 2
_NS = 16
_L = 16

_CHUNK = 16
_GRP = 20
_NGRP = 8
_STRIPE = 624
_PZ = 504

_COLPERM = np.concatenate(
    [np.concatenate([np.arange(0, 32, 2), np.arange(1, 32, 2)]) + 32 * c
     for c in range(4)])


def _sc_accumulate(xb, src_p, tgt_p, w_p, r_total, n):
  d = xb.shape[1] * 2
  npad = n + 8
  rel_per_core = r_total // _NC
  tail = npad - _STRIPE * _NS

  mesh = plsc.VectorSubcoreMesh(core_axis_name="c", subcore_axis_name="s")

  @functools.partial(
      pl.kernel,
      out_type=(
          jax.ShapeDtypeStruct((r_total, npad, d), jnp.float32),
          jax.ShapeDtypeStruct((r_total * npad,), jnp.float32),
      ),
      mesh=mesh,
      compiler_params=pltpu.CompilerParams(use_tc_tiling_on_sc=False),
      scratch_types=[
          pltpu.VMEM((_GRP, _CHUNK), jnp.int32),
          pltpu.VMEM((_GRP, _CHUNK), jnp.int32),
          pltpu.VMEM((_GRP, _CHUNK), jnp.float32),
          pltpu.VMEM((_CHUNK, d // 2), jnp.int32),
          pltpu.VMEM((_CHUNK, d // 2), jnp.int32),
          pltpu.VMEM((_CHUNK, d), jnp.float32),
          pltpu.VMEM((_CHUNK,), jnp.float32),
          pltpu.VMEM((_PZ,), jnp.float32),
          pltpu.VMEM((_PZ,), jnp.float32),
          pltpu.VMEM_SHARED((n, d // 2), jnp.int32),
          pltpu.VMEM_SHARED((npad, d), jnp.float32),
          pltpu.VMEM_SHARED((npad,), jnp.float32),
          pltpu.SemaphoreType.DMA,
          pltpu.SemaphoreType.DMA,
          pltpu.SemaphoreType.DMA,
          pltpu.SemaphoreType.DMA,
      ],
  )
  def sc_kernel(xb_hbm, src_hbm, tgt_hbm, ew_hbm, a_hbm, p_hbm,
                src_g, tgt_g, w_g, rows0_v, rows1_v, pay_v, ones_v, z1_v,
                pv_v, x_sh, a_sh, p_sh, gsem0, gsem1, ssem, psem):
    cid = lax.axis_index("c")
    sid = lax.axis_index("s")
    rows = (rows0_v, rows1_v)
    gsem = (gsem0, gsem1)

    one16 = jnp.full((_L,), 1.0, jnp.float32)
    zero16 = jnp.zeros((_L,), jnp.float32)
    for j in range(_CHUNK // _L):
      ones_v[pl.ds(j * _L, _L)] = one16

    def z1_body(i, carry):
      z1_v[pl.ds(i * _L, _L)] = zero16
      return carry
    lax.fori_loop(0, _PZ // _L, z1_body, 0)
    z1_v[pl.ds(_PZ - _L, _L)] = zero16

    pltpu.sync_copy(xb_hbm.at[pl.ds(sid * _STRIPE, _STRIPE)],
                    x_sh.at[pl.ds(sid * _STRIPE, _STRIPE)])

    @pl.when(sid < 2)
    def _stage_tail():
      half = (n - _STRIPE * _NS) // 2
      pltpu.sync_copy(xb_hbm.at[pl.ds(_STRIPE * _NS + sid * half, half)],
                      x_sh.at[pl.ds(_STRIPE * _NS + sid * half, half)])

    def scale_chunk(buf, w16s):
      def jbody(j, carry):
        w16 = w16s[pl.ds(j * _L, _L)]
        for k in range(_L):
          w = w16[k]
          e = j * _L + k
          for blk in range(d // 32):
            v = buf[e, pl.ds(blk * _L, _L)]
            even = lax.bitcast_convert_type(v << 16, jnp.float32)
            odd = lax.bitcast_convert_type((v >> 16) << 16, jnp.float32)
            pay_v[e, pl.ds(blk * 32, _L)] = even * w
            pay_v[e, pl.ds(blk * 32 + _L, _L)] = odd * w
        return carry
      lax.fori_loop(0, _CHUNK // _L, jbody, 0)

    def rel_body(rr, carry):
      r = cid * rel_per_core + rr
      rt = r * _NS + sid

      def zfill_body(i, carry):
        for c in range(d // _L):
          pay_v[i, pl.ds(c * _L, _L)] = zero16
        return carry
      lax.fori_loop(0, _CHUNK, zfill_body, 0)

      for i in range(_STRIPE // _CHUNK):
        pltpu.sync_copy(pay_v,
                        a_sh.at[pl.ds(sid * _STRIPE + i * _CHUNK, _CHUNK)])
      rem = _STRIPE - (_STRIPE // _CHUNK) * _CHUNK
      if rem:
        pltpu.sync_copy(
            pay_v.at[pl.ds(0, rem)],
            a_sh.at[pl.ds(sid * _STRIPE + _STRIPE - rem, rem)])

      @pl.when(sid < tail // 8)
      def _zero_tail():
        pltpu.sync_copy(pay_v.at[pl.ds(0, 8)],
                        a_sh.at[pl.ds(_STRIPE * _NS + sid * 8, 8)])

      pltpu.sync_copy(z1_v, p_sh.at[pl.ds(sid * _PZ, _PZ)])

      @pl.when(sid < 3)
      def _zero_counts2():
        pltpu.sync_copy(z1_v, p_sh.at[pl.ds((_NS + sid) * _PZ, _PZ)])

      @pl.when(sid == 3)
      def _zero_counts3():
        last = npad - 19 * _PZ
        pltpu.sync_copy(z1_v.at[pl.ds(0, last)],
                        p_sh.at[pl.ds(19 * _PZ, last)])

      plsc.subcore_barrier()

      def grp_body(g, carry):
        pltpu.sync_copy(src_hbm.at[rt, g], src_g)
        pltpu.sync_copy(tgt_hbm.at[rt, g], tgt_g)
        pltpu.sync_copy(ew_hbm.at[rt, g], w_g)

        pltpu.async_copy(x_sh.at[src_g[0]], rows0_v, gsem0)

        def chunk_body(s, carry2):
          for b in range(2):
            c = s * 2 + b
            nxt = 1 - b

            def start_next():
              pltpu.async_copy(x_sh.at[src_g[c + 1]], rows[nxt],
                               gsem[nxt])
            if b == 0:
              start_next()
            else:
              pl.when(s < _GRP // 2 - 1)(start_next)

            pltpu.make_async_copy(x_sh.at[src_g[c]], rows[b],
                                  gsem[b]).wait()

            def wait_prev():
              pltpu.make_async_copy(
                  pay_v, a_sh.at[tgt_g.at[c]], ssem).wait()
              pltpu.make_async_copy(
                  ones_v, p_sh.at[tgt_g.at[c]], psem).wait()
            if b == 0:
              pl.when(s >= 1)(wait_prev)
            else:
              wait_prev()

            scale_chunk(rows[b], w_g.at[c])
            pltpu.async_copy(pay_v, a_sh.at[tgt_g.at[c]], ssem, add=True)
            pltpu.async_copy(ones_v, p_sh.at[tgt_g.at[c]], psem, add=True)
          return carry2
        lax.fori_loop(0, _GRP // 2, chunk_body, 0)

        pltpu.make_async_copy(pay_v, a_sh.at[tgt_g.at[_GRP - 1]],
                              ssem).wait()
        pltpu.make_async_copy(ones_v, p_sh.at[tgt_g.at[_GRP - 1]],
                              psem).wait()
        return carry
      lax.fori_loop(0, _NGRP, grp_body, 0)

      plsc.subcore_barrier()

      pltpu.sync_copy(a_sh.at[pl.ds(sid * _STRIPE, _STRIPE)],
                      a_hbm.at[r, pl.ds(sid * _STRIPE, _STRIPE)])

      @pl.when(sid < tail // 8)
      def _write_tail():
        pltpu.sync_copy(a_sh.at[pl.ds(_STRIPE * _NS + sid * 8, 8)],
                        a_hbm.at[r, pl.ds(_STRIPE * _NS + sid * 8, 8)])

      def wb_counts(stripe, size):
        pltpu.sync_copy(p_sh.at[pl.ds(stripe * _PZ, size)],
                        pv_v.at[pl.ds(0, size)])
        pbase = pl.multiple_of(r * npad + stripe * _PZ, 8)
        pltpu.sync_copy(pv_v.at[pl.ds(0, size)],
                        p_hbm.at[pl.ds(pbase, size)])

      wb_counts(sid, _PZ)

      @pl.when(sid < 3)
      def _wb_counts2():
        wb_counts(_NS + sid, _PZ)

      @pl.when(sid == 3)
      def _wb_counts3():
        wb_counts(19, npad - 19 * _PZ)

      plsc.subcore_barrier()
      return carry

    lax.fori_loop(0, rel_per_core, rel_body, 0)

  return sc_kernel(xb, src_p, tgt_p, w_p)


def _tc_combine(a, counts_t, x, rw_perm, self_weight, bias_param):
  n, d = x.shape
  r_total = rw_perm.shape[0]
  blk = 1000

  def body(a_ref, p_ref, x_ref, rw_ref, sw_ref, b_ref, o_ref):
    acc = jnp.dot(x_ref[...], sw_ref[...], preferred_element_type=jnp.float32)
    for r in range(r_total):
      acc = acc + jnp.dot(a_ref[r], rw_ref[r],
                          preferred_element_type=jnp.float32)
    present = (p_ref[...] > 0).astype(jnp.float32)
    acc = acc + jnp.dot(present, b_ref[...],
                        preferred_element_type=jnp.float32)
    o_ref[...] = acc

  return pl.pallas_call(
      body,
      grid=(n // blk,),
      in_specs=[
          pl.BlockSpec((r_total, blk, d), lambda i: (0, i, 0)),
          pl.BlockSpec((blk, r_total), lambda i: (i, 0)),
          pl.BlockSpec((blk, d), lambda i: (i, 0)),
          pl.BlockSpec((r_total, d, d), lambda i: (0, 0, 0)),
          pl.BlockSpec((d, d), lambda i: (0, 0)),
          pl.BlockSpec((r_total, d), lambda i: (0, 0)),
      ],
      out_specs=pl.BlockSpec((blk, d), lambda i: (i, 0)),
      out_shape=jax.ShapeDtypeStruct((n, d), jnp.float32),
  )(a, counts_t, x, rw_perm, self_weight, bias_param)


def kernel(entity_embeddings, edge_index, edge_weights, relation_weights,
           self_weight, bias_param):
  r_total, _, e_total = edge_index.shape
  n, din = entity_embeddings.shape
  npad = n + 8
  ept = e_total // _NS
  pad = _NGRP * _GRP * _CHUNK - ept

  xb = lax.bitcast_convert_type(
      entity_embeddings.astype(jnp.bfloat16).reshape(n, din // 2, 2),
      jnp.int32)
  src3 = edge_index[:, 0, :].reshape(r_total, _NS, ept)
  tgt3 = edge_index[:, 1, :].reshape(r_total, _NS, ept)
  w3 = edge_weights.reshape(r_total, _NS, ept)
  shp = (r_total * _NS, _NGRP, _GRP, _CHUNK)
  src_p = jnp.pad(src3, ((0, 0), (0, 0), (0, pad))).reshape(shp)
  tgt_p = jnp.pad(tgt3, ((0, 0), (0, 0), (0, pad)),
                  constant_values=n).reshape(shp)
  w_p = jnp.pad(w3, ((0, 0), (0, 0), (0, pad))).reshape(shp)
  rw_perm = relation_weights[:, _COLPERM, :]

  a, counts = _sc_accumulate(xb, src_p, tgt_p, w_p, r_total, n)
  counts_t = counts.reshape(r_total, npad)[:, :n].T
  return _tc_combine(a, counts_t, entity_embeddings, rw_perm,
                     self_weight, bias_param)


# speedup vs baseline: 1.4283x; 1.0002x over previous
import functools

import jax
import jax.numpy as jnp
import numpy as np
from jax import lax
from jax.experimental import pallas as pl
from jax.experimental.pallas import tpu as pltpu
from jax.experimental.pallas import tpu_sc as plsc

_NC = 2
_NS = 16
_L = 16

_CHUNK = 16
_GRP = 20
_NGRP = 8
_STRIPE = 624
_PZ = 504

_COLPERM = np.concatenate(
    [np.concatenate([np.arange(0, 32, 2), np.arange(1, 32, 2)]) + 32 * c
     for c in range(4)])


def _sc_accumulate(xb, src_p, tgt_p, w_p, r_total, n):
  d = xb.shape[1] * 2
  npad = n + 8
  rel_per_core = r_total // _NC
  tail = npad - _STRIPE * _NS

  mesh = plsc.VectorSubcoreMesh(core_axis_name="c", subcore_axis_name="s")

  @functools.partial(
      pl.kernel,
      out_type=(
          jax.ShapeDtypeStruct((r_total, npad, d), jnp.float32),
          jax.ShapeDtypeStruct((r_total * npad,), jnp.float32),
      ),
      mesh=mesh,
      compiler_params=pltpu.CompilerParams(use_tc_tiling_on_sc=False),
      scratch_types=[
          pltpu.VMEM((_GRP, _CHUNK), jnp.int32),
          pltpu.VMEM((_GRP, _CHUNK), jnp.int32),
          pltpu.VMEM((_GRP, _CHUNK), jnp.float32),
          pltpu.VMEM((_CHUNK, d // 2), jnp.int32),
          pltpu.VMEM((_CHUNK, d // 2), jnp.int32),
          pltpu.VMEM((_CHUNK, d), jnp.float32),
          pltpu.VMEM((_CHUNK,), jnp.float32),
          pltpu.VMEM((_PZ,), jnp.float32),
          pltpu.VMEM((_PZ,), jnp.float32),
          pltpu.VMEM_SHARED((n, d // 2), jnp.int32),
          pltpu.VMEM_SHARED((npad, d), jnp.float32),
          pltpu.VMEM_SHARED((npad,), jnp.float32),
          pltpu.SemaphoreType.DMA,
          pltpu.SemaphoreType.DMA,
          pltpu.SemaphoreType.DMA,
          pltpu.SemaphoreType.DMA,
      ],
  )
  def sc_kernel(xb_hbm, src_hbm, tgt_hbm, ew_hbm, a_hbm, p_hbm,
                src_g, tgt_g, w_g, rows0_v, rows1_v, pay_v, ones_v, z1_v,
                pv_v, x_sh, a_sh, p_sh, gsem0, gsem1, ssem, psem):
    cid = lax.axis_index("c")
    sid = lax.axis_index("s")
    rows = (rows0_v, rows1_v)
    gsem = (gsem0, gsem1)

    one16 = jnp.full((_L,), 1.0, jnp.float32)
    zero16 = jnp.zeros((_L,), jnp.float32)
    for j in range(_CHUNK // _L):
      ones_v[pl.ds(j * _L, _L)] = one16

    def z1_body(i, carry):
      z1_v[pl.ds(i * _L, _L)] = zero16
      return carry
    lax.fori_loop(0, _PZ // _L, z1_body, 0)
    z1_v[pl.ds(_PZ - _L, _L)] = zero16

    pltpu.sync_copy(xb_hbm.at[pl.ds(sid * _STRIPE, _STRIPE)],
                    x_sh.at[pl.ds(sid * _STRIPE, _STRIPE)])

    @pl.when(sid < 2)
    def _stage_tail():
      half = (n - _STRIPE * _NS) // 2
      pltpu.sync_copy(xb_hbm.at[pl.ds(_STRIPE * _NS + sid * half, half)],
                      x_sh.at[pl.ds(_STRIPE * _NS + sid * half, half)])

    def scale_chunk(buf, w16s):
      def jbody(j, carry):
        w16 = w16s[pl.ds(j * _L, _L)]
        for k in range(_L):
          w = w16[k]
          e = j * _L + k
          for blk in range(d // 32):
            v = buf[e, pl.ds(blk * _L, _L)]
            even = lax.bitcast_convert_type(v << 16, jnp.float32)
            odd = lax.bitcast_convert_type(
                v & jnp.int32(np.int32(-65536)), jnp.float32)
            pay_v[e, pl.ds(blk * 32, _L)] = even * w
            pay_v[e, pl.ds(blk * 32 + _L, _L)] = odd * w
        return carry
      lax.fori_loop(0, _CHUNK // _L, jbody, 0)

    def rel_body(rr, carry):
      r = cid * rel_per_core + rr
      rt = r * _NS + sid

      def zfill_body(i, carry):
        for c in range(d // _L):
          pay_v[i, pl.ds(c * _L, _L)] = zero16
        return carry
      lax.fori_loop(0, _CHUNK, zfill_body, 0)

      for i in range(_STRIPE // _CHUNK):
        pltpu.sync_copy(pay_v,
                        a_sh.at[pl.ds(sid * _STRIPE + i * _CHUNK, _CHUNK)])
      rem = _STRIPE - (_STRIPE // _CHUNK) * _CHUNK
      if rem:
        pltpu.sync_copy(
            pay_v.at[pl.ds(0, rem)],
            a_sh.at[pl.ds(sid * _STRIPE + _STRIPE - rem, rem)])

      @pl.when(sid < tail // 8)
      def _zero_tail():
        pltpu.sync_copy(pay_v.at[pl.ds(0, 8)],
                        a_sh.at[pl.ds(_STRIPE * _NS + sid * 8, 8)])

      pltpu.sync_copy(z1_v, p_sh.at[pl.ds(sid * _PZ, _PZ)])

      @pl.when(sid < 3)
      def _zero_counts2():
        pltpu.sync_copy(z1_v, p_sh.at[pl.ds((_NS + sid) * _PZ, _PZ)])

      @pl.when(sid == 3)
      def _zero_counts3():
        last = npad - 19 * _PZ
        pltpu.sync_copy(z1_v.at[pl.ds(0, last)],
                        p_sh.at[pl.ds(19 * _PZ, last)])

      plsc.subcore_barrier()

      def grp_body(g, carry):
        pltpu.sync_copy(src_hbm.at[rt, g], src_g)
        pltpu.sync_copy(tgt_hbm.at[rt, g], tgt_g)
        pltpu.sync_copy(ew_hbm.at[rt, g], w_g)

        pltpu.async_copy(x_sh.at[src_g[0]], rows0_v, gsem0)

        def chunk_body(s, carry2):
          for b in range(2):
            c = s * 2 + b
            nxt = 1 - b

            def start_next():
              pltpu.async_copy(x_sh.at[src_g[c + 1]], rows[nxt],
                               gsem[nxt])
            if b == 0:
              start_next()
            else:
              pl.when(s < _GRP // 2 - 1)(start_next)

            pltpu.make_async_copy(x_sh.at[src_g[c]], rows[b],
                                  gsem[b]).wait()

            def wait_prev():
              pltpu.make_async_copy(
                  pay_v, a_sh.at[tgt_g.at[c]], ssem).wait()
              pltpu.make_async_copy(
                  ones_v, p_sh.at[tgt_g.at[c]], psem).wait()
            if b == 0:
              pl.when(s >= 1)(wait_prev)
            else:
              wait_prev()

            scale_chunk(rows[b], w_g.at[c])
            pltpu.async_copy(pay_v, a_sh.at[tgt_g.at[c]], ssem, add=True)
            pltpu.async_copy(ones_v, p_sh.at[tgt_g.at[c]], psem, add=True)
          return carry2
        lax.fori_loop(0, _GRP // 2, chunk_body, 0)

        pltpu.make_async_copy(pay_v, a_sh.at[tgt_g.at[_GRP - 1]],
                              ssem).wait()
        pltpu.make_async_copy(ones_v, p_sh.at[tgt_g.at[_GRP - 1]],
                              psem).wait()
        return carry
      lax.fori_loop(0, _NGRP, grp_body, 0)

      plsc.subcore_barrier()

      pltpu.sync_copy(a_sh.at[pl.ds(sid * _STRIPE, _STRIPE)],
                      a_hbm.at[r, pl.ds(sid * _STRIPE, _STRIPE)])

      @pl.when(sid < tail // 8)
      def _write_tail():
        pltpu.sync_copy(a_sh.at[pl.ds(_STRIPE * _NS + sid * 8, 8)],
                        a_hbm.at[r, pl.ds(_STRIPE * _NS + sid * 8, 8)])

      def wb_counts(stripe, size):
        pltpu.sync_copy(p_sh.at[pl.ds(stripe * _PZ, size)],
                        pv_v.at[pl.ds(0, size)])
        pbase = pl.multiple_of(r * npad + stripe * _PZ, 8)
        pltpu.sync_copy(pv_v.at[pl.ds(0, size)],
                        p_hbm.at[pl.ds(pbase, size)])

      wb_counts(sid, _PZ)

      @pl.when(sid < 3)
      def _wb_counts2():
        wb_counts(_NS + sid, _PZ)

      @pl.when(sid == 3)
      def _wb_counts3():
        wb_counts(19, npad - 19 * _PZ)

      plsc.subcore_barrier()
      return carry

    lax.fori_loop(0, rel_per_core, rel_body, 0)

  return sc_kernel(xb, src_p, tgt_p, w_p)


def _tc_combine(a, counts_t, x, rw_perm, self_weight, bias_param):
  n, d = x.shape
  r_total = rw_perm.shape[0]
  blk = 1000

  def body(a_ref, p_ref, x_ref, rw_ref, sw_ref, b_ref, o_ref):
    acc = jnp.dot(x_ref[...], sw_ref[...], preferred_element_type=jnp.float32)
    for r in range(r_total):
      acc = acc + jnp.dot(a_ref[r], rw_ref[r],
                          preferred_element_type=jnp.float32)
    present = (p_ref[...] > 0).astype(jnp.float32)
    acc = acc + jnp.dot(present, b_ref[...],
                        preferred_element_type=jnp.float32)
    o_ref[...] = acc

  return pl.pallas_call(
      body,
      grid=(n // blk,),
      in_specs=[
          pl.BlockSpec((r_total, blk, d), lambda i: (0, i, 0)),
          pl.BlockSpec((blk, r_total), lambda i: (i, 0)),
          pl.BlockSpec((blk, d), lambda i: (i, 0)),
          pl.BlockSpec((r_total, d, d), lambda i: (0, 0, 0)),
          pl.BlockSpec((d, d), lambda i: (0, 0)),
          pl.BlockSpec((r_total, d), lambda i: (0, 0)),
      ],
      out_specs=pl.BlockSpec((blk, d), lambda i: (i, 0)),
      out_shape=jax.ShapeDtypeStruct((n, d), jnp.float32),
  )(a, counts_t, x, rw_perm, self_weight, bias_param)


def kernel(entity_embeddings, edge_index, edge_weights, relation_weights,
           self_weight, bias_param):
  r_total, _, e_total = edge_index.shape
  n, din = entity_embeddings.shape
  npad = n + 8
  ept = e_total // _NS
  pad = _NGRP * _GRP * _CHUNK - ept

  xb = lax.bitcast_convert_type(
      entity_embeddings.astype(jnp.bfloat16).reshape(n, din // 2, 2),
      jnp.int32)
  src3 = edge_index[:, 0, :].reshape(r_total, _NS, ept)
  tgt3 = edge_index[:, 1, :].reshape(r_total, _NS, ept)
  w3 = edge_weights.reshape(r_total, _NS, ept)
  shp = (r_total * _NS, _NGRP, _GRP, _CHUNK)
  src_p = jnp.pad(src3, ((0, 0), (0, 0), (0, pad))).reshape(shp)
  tgt_p = jnp.pad(tgt3, ((0, 0), (0, 0), (0, pad)),
                  constant_values=n).reshape(shp)
  w_p = jnp.pad(w3, ((0, 0), (0, 0), (0, pad))).reshape(shp)
  rw_perm = relation_weights[:, _COLPERM, :]

  a, counts = _sc_accumulate(xb, src_p, tgt_p, w_p, r_total, n)
  counts_t = counts.reshape(r_total, npad)[:, :n].T
  return _tc_combine(a, counts_t, entity_embeddings, rw_perm,
                     self_weight, bias_param)


# speedup vs baseline: 1.6462x; 1.1526x over previous
import functools

import jax
import jax.numpy as jnp
import numpy as np
from jax import lax
from jax.experimental import pallas as pl
from jax.experimental.pallas import tpu as pltpu
from jax.experimental.pallas import tpu_sc as plsc

_NC = 2
_NS = 16
_L = 16

_CHUNK = 16
_GRP = 20
_NGRP = 8
_EPT = _NGRP * _GRP * _CHUNK
_STRIPE = 624
_PZ = 504

_COLPERM = np.concatenate(
    [np.concatenate([np.arange(0, 32, 2), np.arange(1, 32, 2)]) + 32 * c
     for c in range(4)])


def _sc_accumulate(xb, ei_p, ew_p, r_total, n):
  d = xb.shape[1] * 2
  npad = n + 8
  rel_per_core = r_total // _NC
  tail = npad - _STRIPE * _NS

  mesh = plsc.VectorSubcoreMesh(core_axis_name="c", subcore_axis_name="s")

  @functools.partial(
      pl.kernel,
      out_type=(
          jax.ShapeDtypeStruct((r_total, npad, d), jnp.float32),
          jax.ShapeDtypeStruct((r_total * npad,), jnp.float32),
      ),
      mesh=mesh,
      compiler_params=pltpu.CompilerParams(use_tc_tiling_on_sc=False),
      scratch_types=[
          pltpu.VMEM((_GRP, _CHUNK), jnp.int32),
          pltpu.VMEM((_GRP, _CHUNK), jnp.int32),
          pltpu.VMEM((_GRP, _CHUNK), jnp.float32),
          pltpu.VMEM((_CHUNK, d // 2), jnp.int32),
          pltpu.VMEM((_CHUNK, d // 2), jnp.int32),
          pltpu.VMEM((_CHUNK, d), jnp.float32),
          pltpu.VMEM((_CHUNK,), jnp.float32),
          pltpu.VMEM((_PZ,), jnp.float32),
          pltpu.VMEM((_PZ,), jnp.float32),
          pltpu.VMEM_SHARED((n, d // 2), jnp.int32),
          pltpu.VMEM_SHARED((npad, d), jnp.float32),
          pltpu.VMEM_SHARED((npad,), jnp.float32),
          pltpu.SemaphoreType.DMA,
          pltpu.SemaphoreType.DMA,
          pltpu.SemaphoreType.DMA,
          pltpu.SemaphoreType.DMA,
      ],
  )
  def sc_kernel(xb_hbm, ei_hbm, ew_hbm, a_hbm, p_hbm,
                src_l, tgt_l, w_l, rows0_v, rows1_v, pay_v, ones_v, z1_v,
                pv_v, x_sh, a_sh, p_sh, gsem0, gsem1, ssem, psem):
    cid = lax.axis_index("c")
    sid = lax.axis_index("s")
    rows = (rows0_v, rows1_v)
    gsem = (gsem0, gsem1)

    one16 = jnp.full((_L,), 1.0, jnp.float32)
    zero16 = jnp.zeros((_L,), jnp.float32)
    for j in range(_CHUNK // _L):
      ones_v[pl.ds(j * _L, _L)] = one16

    def z1_body(i, carry):
      z1_v[pl.ds(i * _L, _L)] = zero16
      return carry
    lax.fori_loop(0, _PZ // _L, z1_body, 0)
    z1_v[pl.ds(_PZ - _L, _L)] = zero16

    pltpu.sync_copy(xb_hbm.at[pl.ds(sid * _STRIPE, _STRIPE)],
                    x_sh.at[pl.ds(sid * _STRIPE, _STRIPE)])

    @pl.when(sid < 2)
    def _stage_tail():
      half = (n - _STRIPE * _NS) // 2
      pltpu.sync_copy(xb_hbm.at[pl.ds(_STRIPE * _NS + sid * half, half)],
                      x_sh.at[pl.ds(_STRIPE * _NS + sid * half, half)])

    def scale_chunk(buf, w16s):
      def jbody(j, carry):
        w16 = w16s[pl.ds(j * _L, _L)]
        for k in range(_L):
          w = w16[k]
          e = j * _L + k
          for blk in range(d // 32):
            v = buf[e, pl.ds(blk * _L, _L)]
            even = lax.bitcast_convert_type(v << 16, jnp.float32)
            odd = lax.bitcast_convert_type(
                v & jnp.int32(np.int32(-65536)), jnp.float32)
            pay_v[e, pl.ds(blk * 32, _L)] = even * w
            pay_v[e, pl.ds(blk * 32 + _L, _L)] = odd * w
        return carry
      lax.fori_loop(0, _CHUNK // _L, jbody, 0)

    def rel_body(rr, carry):
      r = cid * rel_per_core + rr

      def zfill_body(i, carry):
        for c in range(d // _L):
          pay_v[i, pl.ds(c * _L, _L)] = zero16
        return carry
      lax.fori_loop(0, _CHUNK, zfill_body, 0)

      for i in range(_STRIPE // _CHUNK):
        pltpu.sync_copy(pay_v,
                        a_sh.at[pl.ds(sid * _STRIPE + i * _CHUNK, _CHUNK)])
      rem = _STRIPE - (_STRIPE // _CHUNK) * _CHUNK
      if rem:
        pltpu.sync_copy(
            pay_v.at[pl.ds(0, rem)],
            a_sh.at[pl.ds(sid * _STRIPE + _STRIPE - rem, rem)])

      @pl.when(sid < tail // 8)
      def _zero_tail():
        pltpu.sync_copy(pay_v.at[pl.ds(0, 8)],
                        a_sh.at[pl.ds(_STRIPE * _NS + sid * 8, 8)])

      pltpu.sync_copy(z1_v, p_sh.at[pl.ds(sid * _PZ, _PZ)])

      @pl.when(sid < 3)
      def _zero_counts2():
        pltpu.sync_copy(z1_v, p_sh.at[pl.ds((_NS + sid) * _PZ, _PZ)])

      @pl.when(sid == 3)
      def _zero_counts3():
        last = npad - 19 * _PZ
        pltpu.sync_copy(z1_v.at[pl.ds(0, last)],
                        p_sh.at[pl.ds(19 * _PZ, last)])

      plsc.subcore_barrier()

      def grp_body(g, carry):
        pltpu.sync_copy(ei_hbm.at[r, 0, sid, g], src_l)
        pltpu.sync_copy(ei_hbm.at[r, 1, sid, g], tgt_l)
        pltpu.sync_copy(ew_hbm.at[r, sid, g], w_l)

        pltpu.async_copy(x_sh.at[jnp.minimum(src_l[0], n - 1)],
                         rows0_v, gsem0)

        def chunk_body(s, carry2):
          for b in range(2):
            c = s * 2 + b
            nxt = 1 - b

            def start_next():
              pltpu.async_copy(x_sh.at[jnp.minimum(src_l[c + 1], n - 1)],
                               rows[nxt], gsem[nxt])
            if b == 0:
              start_next()
            else:
              pl.when(s < _GRP // 2 - 1)(start_next)

            pltpu.make_async_copy(x_sh.at[jnp.minimum(src_l[c], n - 1)],
                                  rows[b], gsem[b]).wait()

            def wait_prev():
              pltpu.make_async_copy(
                  pay_v, a_sh.at[tgt_l.at[c]], ssem).wait()
              pltpu.make_async_copy(
                  ones_v, p_sh.at[tgt_l.at[c]], psem).wait()
            if b == 0:
              pl.when(s >= 1)(wait_prev)
            else:
              wait_prev()

            scale_chunk(rows[b], w_l.at[c])
            pltpu.async_copy(pay_v, a_sh.at[tgt_l.at[c]], ssem, add=True)
            pltpu.async_copy(ones_v, p_sh.at[tgt_l.at[c]], psem, add=True)
          return carry2
        lax.fori_loop(0, _GRP // 2, chunk_body, 0)

        pltpu.make_async_copy(pay_v, a_sh.at[tgt_l.at[_GRP - 1]],
                              ssem).wait()
        pltpu.make_async_copy(ones_v, p_sh.at[tgt_l.at[_GRP - 1]],
                              psem).wait()
        return carry
      lax.fori_loop(0, _NGRP, grp_body, 0)

      plsc.subcore_barrier()

      pltpu.sync_copy(a_sh.at[pl.ds(sid * _STRIPE, _STRIPE)],
                      a_hbm.at[r, pl.ds(sid * _STRIPE, _STRIPE)])

      @pl.when(sid < tail // 8)
      def _write_tail():
        pltpu.sync_copy(a_sh.at[pl.ds(_STRIPE * _NS + sid * 8, 8)],
                        a_hbm.at[r, pl.ds(_STRIPE * _NS + sid * 8, 8)])

      def wb_counts(stripe, size):
        pltpu.sync_copy(p_sh.at[pl.ds(stripe * _PZ, size)],
                        pv_v.at[pl.ds(0, size)])
        pbase = pl.multiple_of(r * npad + stripe * _PZ, 8)
        pltpu.sync_copy(pv_v.at[pl.ds(0, size)],
                        p_hbm.at[pl.ds(pbase, size)])

      wb_counts(sid, _PZ)

      @pl.when(sid < 3)
      def _wb_counts2():
        wb_counts(_NS + sid, _PZ)

      @pl.when(sid == 3)
      def _wb_counts3():
        wb_counts(19, npad - 19 * _PZ)

      plsc.subcore_barrier()
      return carry

    lax.fori_loop(0, rel_per_core, rel_body, 0)

  return sc_kernel(xb, ei_p, ew_p)


def _tc_combine(a, counts_t, x, rw_perm, self_weight, bias_param):
  n, d = x.shape
  r_total = rw_perm.shape[0]
  blk = 1000

  def body(a_ref, p_ref, x_ref, rw_ref, sw_ref, b_ref, o_ref):
    acc = jnp.dot(x_ref[...], sw_ref[...], preferred_element_type=jnp.float32)
    for r in range(r_total):
      acc = acc + jnp.dot(a_ref[r], rw_ref[r],
                          preferred_element_type=jnp.float32)
    present = (p_ref[...] > 0).astype(jnp.float32)
    acc = acc + jnp.dot(present, b_ref[...],
                        preferred_element_type=jnp.float32)
    o_ref[...] = acc

  return pl.pallas_call(
      body,
      grid=(n // blk,),
      in_specs=[
          pl.BlockSpec((r_total, blk, d), lambda i: (0, i, 0)),
          pl.BlockSpec((blk, r_total), lambda i: (i, 0)),
          pl.BlockSpec((blk, d), lambda i: (i, 0)),
          pl.BlockSpec((r_total, d, d), lambda i: (0, 0, 0)),
          pl.BlockSpec((d, d), lambda i: (0, 0)),
          pl.BlockSpec((r_total, d), lambda i: (0, 0)),
      ],
      out_specs=pl.BlockSpec((blk, d), lambda i: (i, 0)),
      out_shape=jax.ShapeDtypeStruct((n, d), jnp.float32),
  )(a, counts_t, x, rw_perm, self_weight, bias_param)


def kernel(entity_embeddings, edge_index, edge_weights, relation_weights,
           self_weight, bias_param):
  r_total, _, e_total = edge_index.shape
  n, din = entity_embeddings.shape
  npad = n + 8
  pad = _NS * _EPT - e_total

  xb = lax.bitcast_convert_type(
      entity_embeddings.astype(jnp.bfloat16).reshape(n, din // 2, 2),
      jnp.int32)
  ei_p = jnp.pad(edge_index, ((0, 0), (0, 0), (0, pad)),
                 constant_values=n).reshape(
                     r_total, 2, _NS, _NGRP, _GRP, _CHUNK)
  ew_p = jnp.pad(edge_weights, ((0, 0), (0, pad))).reshape(
      r_total, _NS, _NGRP, _GRP, _CHUNK)
  rw_perm = relation_weights[:, _COLPERM, :]

  a, counts = _sc_accumulate(xb, ei_p, ew_p, r_total, n)
  counts_t = counts.reshape(r_total, npad)[:, :n].T
  return _tc_combine(a, counts_t, entity_embeddings, rw_perm,
                     self_weight, bias_param)
